# initial kernel scaffold (unmeasured)
import jax
import jax.numpy as jnp
from jax import lax
from jax.experimental import pallas as pl
from jax.experimental.pallas import tpu as pltpu

N_DEV = 8
N_HOPS = 4


def kernel(x, w_mat):
    x = x.astype(jnp.bfloat16)
    w_mat = w_mat.astype(jnp.bfloat16)

    m_per, k = x.shape
    _, n_per = w_mat.shape
    half = m_per // 2

    def body(x_ref, w_ref, out_ref, cw_buf, ccw_buf,
             cw_send, cw_recv, ccw_send, ccw_recv):
        me = lax.axis_index("i")
        right = lax.rem(me + 1, N_DEV)
        left = lax.rem(me + N_DEV - 1, N_DEV)

        barrier_sem = pltpu.get_barrier_semaphore()
        for nbr in (left, right):
            pl.semaphore_signal(
                barrier_sem, inc=1,
                device_id=(nbr,), device_id_type=pl.DeviceIdType.MESH,
            )
        pl.semaphore_wait(barrier_sem, 2)

        def gemm(chunk):
            return lax.dot_general(
                chunk, w_ref[...],
                (((1,), (0,)), ((), ())),
                preferred_element_type=jnp.float32,
            )

        started = []

        cw0 = pltpu.make_async_remote_copy(
            src_ref=x_ref, dst_ref=cw_buf.at[0],
            send_sem=cw_send.at[0], recv_sem=cw_recv.at[0],
            device_id=(right,), device_id_type=pl.DeviceIdType.MESH,
        )
        cw0.start()
        started.append(cw0)
        ccw0 = pltpu.make_async_remote_copy(
            src_ref=x_ref, dst_ref=ccw_buf.at[0],
            send_sem=ccw_send.at[0], recv_sem=ccw_recv.at[0],
            device_id=(left,), device_id_type=pl.DeviceIdType.MESH,
        )
        ccw0.start()
        started.append(ccw0)

        out_ref[pl.ds(me * m_per, m_per), :] = gemm(x_ref[...])

        for h in range(N_HOPS):
            recv_cw = pltpu.make_async_remote_copy(
                src_ref=cw_buf.at[h], dst_ref=cw_buf.at[h],
                send_sem=cw_send.at[h], recv_sem=cw_recv.at[h],
                device_id=(right,), device_id_type=pl.DeviceIdType.MESH,
            )
            recv_cw.wait_recv()
            if h < N_HOPS - 1:
                if h + 1 == N_HOPS - 1:
                    src = cw_buf.at[h, pl.ds(0, half)]
                    dst = cw_buf.at[h + 1, pl.ds(0, half)]
                else:
                    src = cw_buf.at[h]
                    dst = cw_buf.at[h + 1]
                fwd = pltpu.make_async_remote_copy(
                    src_ref=src, dst_ref=dst,
                    send_sem=cw_send.at[h + 1], recv_sem=cw_recv.at[h + 1],
                    device_id=(right,), device_id_type=pl.DeviceIdType.MESH,
                )
                fwd.start()
                started.append(fwd)

            recv_ccw = pltpu.make_async_remote_copy(
                src_ref=ccw_buf.at[h], dst_ref=ccw_buf.at[h],
                send_sem=ccw_send.at[h], recv_sem=ccw_recv.at[h],
                device_id=(left,), device_id_type=pl.DeviceIdType.MESH,
            )
            recv_ccw.wait_recv()
            if h < N_HOPS - 1:
                if h + 1 == N_HOPS - 1:
                    src = ccw_buf.at[h, pl.ds(half, half)]
                    dst = ccw_buf.at[h + 1, pl.ds(half, half)]
                else:
                    src = ccw_buf.at[h]
                    dst = ccw_buf.at[h + 1]
                fwd = pltpu.make_async_remote_copy(
                    src_ref=src, dst_ref=dst,
                    send_sem=ccw_send.at[h + 1], recv_sem=ccw_recv.at[h + 1],
                    device_id=(left,), device_id_type=pl.DeviceIdType.MESH,
                )
                fwd.start()
                started.append(fwd)

            origin_cw = lax.rem(me + N_DEV - h - 1, N_DEV)
            origin_ccw = lax.rem(me + h + 1, N_DEV)
            if h < N_HOPS - 1:
                out_ref[pl.ds(origin_cw * m_per, m_per), :] = gemm(
                    cw_buf[h])
                out_ref[pl.ds(origin_ccw * m_per, m_per), :] = gemm(
                    ccw_buf[h])
            else:
                out_ref[pl.ds(origin_cw * m_per, half), :] = gemm(
                    cw_buf[h, pl.ds(0, half)])
                out_ref[pl.ds(origin_ccw * m_per + half, half), :] = gemm(
                    ccw_buf[h, pl.ds(half, half)])

        for s in started:
            s.wait_send()

    out_shape = jax.ShapeDtypeStruct((N_DEV * m_per, n_per), jnp.float32)
    return pl.pallas_call(
        body,
        out_shape=out_shape,
        in_specs=[
            pl.BlockSpec(memory_space=pltpu.VMEM),
            pl.BlockSpec(memory_space=pltpu.VMEM),
        ],
        out_specs=pl.BlockSpec(memory_space=pltpu.VMEM),
        scratch_shapes=[
            pltpu.VMEM((N_HOPS, m_per, k), jnp.bfloat16),
            pltpu.VMEM((N_HOPS, m_per, k), jnp.bfloat16),
            pltpu.SemaphoreType.DMA((N_HOPS,)),
            pltpu.SemaphoreType.DMA((N_HOPS,)),
            pltpu.SemaphoreType.DMA((N_HOPS,)),
            pltpu.SemaphoreType.DMA((N_HOPS,)),
        ],
        compiler_params=pltpu.CompilerParams(collective_id=0),
    )(x, w_mat)


# baseline (device time: 211336 ns/iter reference)
import jax
import jax.numpy as jnp
from jax import lax
from jax.experimental import pallas as pl
from jax.experimental.pallas import tpu as pltpu

N_DEV = 8
N_HOPS = 4


def kernel(x, w_mat):
    x = x.astype(jnp.bfloat16)
    w_mat = w_mat.astype(jnp.bfloat16)

    m_per, k = x.shape
    _, n_per = w_mat.shape
    half = m_per // 2

    def body(x_ref, w_ref, out_ref, cw_buf, ccw_buf,
             cw_send, cw_recv, ccw_send, ccw_recv):
        me = lax.axis_index("i")
        right = lax.rem(me + 1, N_DEV)
        left = lax.rem(me + N_DEV - 1, N_DEV)

        barrier_sem = pltpu.get_barrier_semaphore()
        for nbr in (left, right):
            pl.semaphore_signal(
                barrier_sem, inc=1,
                device_id=(nbr,), device_id_type=pl.DeviceIdType.MESH,
            )
        pl.semaphore_wait(barrier_sem, 2)

        def gemm(chunk):
            return lax.dot_general(
                chunk, w_ref[...],
                (((1,), (0,)), ((), ())),
                preferred_element_type=jnp.float32,
            )

        started = []

        cw0 = pltpu.make_async_remote_copy(
            src_ref=x_ref, dst_ref=cw_buf.at[0],
            send_sem=cw_send.at[0], recv_sem=cw_recv.at[0],
            device_id=(right,), device_id_type=pl.DeviceIdType.MESH,
        )
        cw0.start()
        started.append(cw0)
        ccw0 = pltpu.make_async_remote_copy(
            src_ref=x_ref, dst_ref=ccw_buf.at[0],
            send_sem=ccw_send.at[0], recv_sem=ccw_recv.at[0],
            device_id=(left,), device_id_type=pl.DeviceIdType.MESH,
        )
        ccw0.start()
        started.append(ccw0)

        out_ref[pl.ds(me * m_per, m_per), :] = gemm(x_ref[...])

        for h in range(N_HOPS):
            recv_cw = pltpu.make_async_remote_copy(
                src_ref=cw_buf.at[h], dst_ref=cw_buf.at[h],
                send_sem=cw_send.at[h], recv_sem=cw_recv.at[h],
                device_id=(right,), device_id_type=pl.DeviceIdType.MESH,
            )
            recv_cw.wait_recv()
            if h < N_HOPS - 1:
                fwd = pltpu.make_async_remote_copy(
                    src_ref=cw_buf.at[h], dst_ref=cw_buf.at[h + 1],
                    send_sem=cw_send.at[h + 1], recv_sem=cw_recv.at[h + 1],
                    device_id=(right,), device_id_type=pl.DeviceIdType.MESH,
                )
                fwd.start()
                started.append(fwd)

            recv_ccw = pltpu.make_async_remote_copy(
                src_ref=ccw_buf.at[h], dst_ref=ccw_buf.at[h],
                send_sem=ccw_send.at[h], recv_sem=ccw_recv.at[h],
                device_id=(left,), device_id_type=pl.DeviceIdType.MESH,
            )
            recv_ccw.wait_recv()
            if h < N_HOPS - 1:
                fwd = pltpu.make_async_remote_copy(
                    src_ref=ccw_buf.at[h], dst_ref=ccw_buf.at[h + 1],
                    send_sem=ccw_send.at[h + 1], recv_sem=ccw_recv.at[h + 1],
                    device_id=(left,), device_id_type=pl.DeviceIdType.MESH,
                )
                fwd.start()
                started.append(fwd)

            origin_cw = lax.rem(me + N_DEV - h - 1, N_DEV)
            origin_ccw = lax.rem(me + h + 1, N_DEV)
            if h < N_HOPS - 1:
                out_ref[pl.ds(origin_cw * m_per, m_per), :] = gemm(
                    cw_buf[h])
                out_ref[pl.ds(origin_ccw * m_per, m_per), :] = gemm(
                    ccw_buf[h])
            else:
                out_ref[pl.ds(origin_cw * m_per, half), :] = gemm(
                    cw_buf[h, pl.ds(0, half)])
                out_ref[pl.ds(origin_ccw * m_per + half, half), :] = gemm(
                    ccw_buf[h, pl.ds(half, half)])

        for s in started:
            s.wait_send()

    out_shape = jax.ShapeDtypeStruct((N_DEV * m_per, n_per), jnp.float32)
    return pl.pallas_call(
        body,
        out_shape=out_shape,
        in_specs=[
            pl.BlockSpec(memory_space=pltpu.VMEM),
            pl.BlockSpec(memory_space=pltpu.VMEM),
        ],
        out_specs=pl.BlockSpec(memory_space=pltpu.VMEM),
        scratch_shapes=[
            pltpu.VMEM((N_HOPS, m_per, k), jnp.bfloat16),
            pltpu.VMEM((N_HOPS, m_per, k), jnp.bfloat16),
            pltpu.SemaphoreType.DMA((N_HOPS,)),
            pltpu.SemaphoreType.DMA((N_HOPS,)),
            pltpu.SemaphoreType.DMA((N_HOPS,)),
            pltpu.SemaphoreType.DMA((N_HOPS,)),
        ],
        compiler_params=pltpu.CompilerParams(
            collective_id=0,
            vmem_limit_bytes=100 * 1024 * 1024,
        ),
    )(x, w_mat)


# device time: 183286 ns/iter; 1.1530x vs baseline; 1.1530x over previous
import jax
import jax.numpy as jnp
from jax import lax
from jax.experimental import pallas as pl
from jax.experimental.pallas import tpu as pltpu

N_DEV = 8
N_PIECES = 7


def kernel(x, w_mat):
    m_per, k = x.shape
    _, n_per = w_mat.shape
    half = m_per // 2
    x = x.astype(jnp.bfloat16).reshape(2, half, k)
    w_mat = w_mat.astype(jnp.bfloat16)

    def body(x_ref, w_ref, out_ref, cw_buf, ccw_buf,
             cw_send, cw_recv, ccw_send, ccw_recv):
        me = lax.axis_index("i")
        right = lax.rem(me + 1, N_DEV)
        left = lax.rem(me + N_DEV - 1, N_DEV)

        barrier_sem = pltpu.get_barrier_semaphore()
        for nbr in (left, right):
            pl.semaphore_signal(
                barrier_sem, inc=1,
                device_id=(nbr,), device_id_type=pl.DeviceIdType.MESH,
            )
        pl.semaphore_wait(barrier_sem, 2)

        def gemm(piece):
            return lax.dot_general(
                piece, w_ref[...],
                (((1,), (0,)), ((), ())),
                preferred_element_type=jnp.float32,
            )

        def send(src, dst_buf, slot, send_sems, recv_sems, target):
            rdma = pltpu.make_async_remote_copy(
                src_ref=src, dst_ref=dst_buf.at[slot],
                send_sem=send_sems.at[slot], recv_sem=recv_sems.at[slot],
                device_id=(target,), device_id_type=pl.DeviceIdType.MESH,
            )
            rdma.start()
            return rdma

        started = []
        started.append(send(x_ref.at[0], cw_buf, 0, cw_send, cw_recv, right))
        started.append(send(x_ref.at[1], cw_buf, 1, cw_send, cw_recv, right))
        started.append(send(x_ref.at[1], ccw_buf, 0, ccw_send, ccw_recv, left))
        started.append(send(x_ref.at[0], ccw_buf, 1, ccw_send, ccw_recv, left))

        out_ref[pl.ds(me * m_per, half), :] = gemm(x_ref[0])
        out_ref[pl.ds(me * m_per + half, half), :] = gemm(x_ref[1])

        def wait_recv(buf, send_sems, recv_sems, slot, peer):
            rdma = pltpu.make_async_remote_copy(
                src_ref=buf.at[slot], dst_ref=buf.at[slot],
                send_sem=send_sems.at[slot], recv_sem=recv_sems.at[slot],
                device_id=(peer,), device_id_type=pl.DeviceIdType.MESH,
            )
            rdma.wait_recv()

        for s in range(N_PIECES):
            wait_recv(cw_buf, cw_send, cw_recv, s, right)
            if s + 2 < N_PIECES:
                started.append(
                    send(cw_buf.at[s], cw_buf, s + 2, cw_send, cw_recv, right))

            wait_recv(ccw_buf, ccw_send, ccw_recv, s, left)
            if s + 2 < N_PIECES:
                started.append(
                    send(ccw_buf.at[s], ccw_buf, s + 2, ccw_send, ccw_recv,
                         left))

            dist = s // 2 + 1
            origin_cw = lax.rem(me + N_DEV - dist, N_DEV)
            origin_ccw = lax.rem(me + dist, N_DEV)
            cw_off = 0 if s % 2 == 0 else half
            ccw_off = half if s % 2 == 0 else 0
            out_ref[pl.ds(origin_cw * m_per + cw_off, half), :] = gemm(
                cw_buf[s])
            out_ref[pl.ds(origin_ccw * m_per + ccw_off, half), :] = gemm(
                ccw_buf[s])

        for r in started:
            r.wait_send()

    out_shape = jax.ShapeDtypeStruct((N_DEV * m_per, n_per), jnp.float32)
    return pl.pallas_call(
        body,
        out_shape=out_shape,
        in_specs=[
            pl.BlockSpec(memory_space=pltpu.VMEM),
            pl.BlockSpec(memory_space=pltpu.VMEM),
        ],
        out_specs=pl.BlockSpec(memory_space=pltpu.VMEM),
        scratch_shapes=[
            pltpu.VMEM((N_PIECES, half, k), jnp.bfloat16),
            pltpu.VMEM((N_PIECES, half, k), jnp.bfloat16),
            pltpu.SemaphoreType.DMA((N_PIECES,)),
            pltpu.SemaphoreType.DMA((N_PIECES,)),
            pltpu.SemaphoreType.DMA((N_PIECES,)),
            pltpu.SemaphoreType.DMA((N_PIECES,)),
        ],
        compiler_params=pltpu.CompilerParams(
            collective_id=0,
            vmem_limit_bytes=100 * 1024 * 1024,
        ),
    )(x, w_mat)


# device time: 175980 ns/iter; 1.2009x vs baseline; 1.0415x over previous
import jax
import jax.numpy as jnp
from jax import lax
from jax.experimental import pallas as pl
from jax.experimental.pallas import tpu as pltpu

N_DEV = 8
N_PIECES = 7


def kernel(x, w_mat):
    m_per, k = x.shape
    _, n_per = w_mat.shape
    half = m_per // 2
    x = x.reshape(2, half, k)

    def body(x_ref, w_ref, out_ref, own_buf, w_bf, cw_buf, ccw_buf,
             cw_send, cw_recv, ccw_send, ccw_recv):
        me = lax.axis_index("i")
        right = lax.rem(me + 1, N_DEV)
        left = lax.rem(me + N_DEV - 1, N_DEV)

        barrier_sem = pltpu.get_barrier_semaphore()
        for nbr in (left, right):
            pl.semaphore_signal(
                barrier_sem, inc=1,
                device_id=(nbr,), device_id_type=pl.DeviceIdType.MESH,
            )
        own_buf[0] = x_ref[0].astype(jnp.bfloat16)
        own_buf[1] = x_ref[1].astype(jnp.bfloat16)
        w_bf[...] = w_ref[...].astype(jnp.bfloat16)
        pl.semaphore_wait(barrier_sem, 2)

        def gemm(piece):
            return lax.dot_general(
                piece, w_bf[...],
                (((1,), (0,)), ((), ())),
                preferred_element_type=jnp.float32,
            )

        def send(src, dst_buf, slot, send_sems, recv_sems, target):
            rdma = pltpu.make_async_remote_copy(
                src_ref=src, dst_ref=dst_buf.at[slot],
                send_sem=send_sems.at[slot], recv_sem=recv_sems.at[slot],
                device_id=(target,), device_id_type=pl.DeviceIdType.MESH,
            )
            rdma.start()
            return rdma

        started = []
        started.append(send(own_buf.at[0], cw_buf, 0, cw_send, cw_recv, right))
        started.append(send(own_buf.at[1], cw_buf, 1, cw_send, cw_recv, right))
        started.append(send(own_buf.at[1], ccw_buf, 0, ccw_send, ccw_recv, left))
        started.append(send(own_buf.at[0], ccw_buf, 1, ccw_send, ccw_recv, left))

        out_ref[pl.ds(me * m_per, half), :] = gemm(own_buf[0])
        out_ref[pl.ds(me * m_per + half, half), :] = gemm(own_buf[1])

        def wait_recv(buf, send_sems, recv_sems, slot, peer):
            rdma = pltpu.make_async_remote_copy(
                src_ref=buf.at[slot], dst_ref=buf.at[slot],
                send_sem=send_sems.at[slot], recv_sem=recv_sems.at[slot],
                device_id=(peer,), device_id_type=pl.DeviceIdType.MESH,
            )
            rdma.wait_recv()

        for s in range(N_PIECES):
            wait_recv(cw_buf, cw_send, cw_recv, s, right)
            if s + 2 < N_PIECES:
                started.append(
                    send(cw_buf.at[s], cw_buf, s + 2, cw_send, cw_recv, right))

            wait_recv(ccw_buf, ccw_send, ccw_recv, s, left)
            if s + 2 < N_PIECES:
                started.append(
                    send(ccw_buf.at[s], ccw_buf, s + 2, ccw_send, ccw_recv,
                         left))

            dist = s // 2 + 1
            origin_cw = lax.rem(me + N_DEV - dist, N_DEV)
            origin_ccw = lax.rem(me + dist, N_DEV)
            cw_off = 0 if s % 2 == 0 else half
            ccw_off = half if s % 2 == 0 else 0
            out_ref[pl.ds(origin_cw * m_per + cw_off, half), :] = gemm(
                cw_buf[s])
            out_ref[pl.ds(origin_ccw * m_per + ccw_off, half), :] = gemm(
                ccw_buf[s])

        for r in started:
            r.wait_send()

    out_shape = jax.ShapeDtypeStruct((N_DEV * m_per, n_per), jnp.float32)
    return pl.pallas_call(
        body,
        out_shape=out_shape,
        in_specs=[
            pl.BlockSpec(memory_space=pltpu.VMEM),
            pl.BlockSpec(memory_space=pltpu.VMEM),
        ],
        out_specs=pl.BlockSpec(memory_space=pltpu.VMEM),
        scratch_shapes=[
            pltpu.VMEM((2, half, k), jnp.bfloat16),
            pltpu.VMEM((k, n_per), jnp.bfloat16),
            pltpu.VMEM((N_PIECES, half, k), jnp.bfloat16),
            pltpu.VMEM((N_PIECES, half, k), jnp.bfloat16),
            pltpu.SemaphoreType.DMA((N_PIECES,)),
            pltpu.SemaphoreType.DMA((N_PIECES,)),
            pltpu.SemaphoreType.DMA((N_PIECES,)),
            pltpu.SemaphoreType.DMA((N_PIECES,)),
        ],
        compiler_params=pltpu.CompilerParams(
            collective_id=0,
            vmem_limit_bytes=100 * 1024 * 1024,
        ),
    )(x, w_mat)


# device time: 175080 ns/iter; 1.2071x vs baseline; 1.0051x over previous
import jax
import jax.numpy as jnp
from jax import lax
from jax.experimental import pallas as pl
from jax.experimental.pallas import tpu as pltpu

N_DEV = 8
N_PIECES = 7


def kernel(x, w_mat):
    m_per, k = x.shape
    _, n_per = w_mat.shape
    half = m_per // 2
    x = x.reshape(2, half, k)

    def body(x_ref, w_ref, out_ref, own_buf, w_bf, cw_buf, ccw_buf,
             cw_send, cw_recv, ccw_send, ccw_recv):
        me = lax.axis_index("i")

        def pos_of_ring(r):
            r = lax.rem(r + N_DEV, N_DEV)
            return jnp.where(r < 4, r, 11 - r)

        my_r = jnp.where(me < 4, me, 11 - me)
        right = pos_of_ring(my_r + 1)
        left = pos_of_ring(my_r - 1)

        barrier_sem = pltpu.get_barrier_semaphore()
        for nbr in (left, right):
            pl.semaphore_signal(
                barrier_sem, inc=1,
                device_id=(nbr,), device_id_type=pl.DeviceIdType.MESH,
            )
        own_buf[0] = x_ref[0].astype(jnp.bfloat16)
        own_buf[1] = x_ref[1].astype(jnp.bfloat16)
        w_bf[...] = w_ref[...].astype(jnp.bfloat16)
        pl.semaphore_wait(barrier_sem, 2)

        def gemm(piece):
            return lax.dot_general(
                piece, w_bf[...],
                (((1,), (0,)), ((), ())),
                preferred_element_type=jnp.float32,
            )

        def send(src, dst_buf, slot, send_sems, recv_sems, target):
            rdma = pltpu.make_async_remote_copy(
                src_ref=src, dst_ref=dst_buf.at[slot],
                send_sem=send_sems.at[slot], recv_sem=recv_sems.at[slot],
                device_id=(target,), device_id_type=pl.DeviceIdType.MESH,
            )
            rdma.start()
            return rdma

        started = []
        started.append(send(own_buf.at[0], cw_buf, 0, cw_send, cw_recv, right))
        started.append(send(own_buf.at[1], cw_buf, 1, cw_send, cw_recv, right))
        started.append(send(own_buf.at[1], ccw_buf, 0, ccw_send, ccw_recv, left))
        started.append(send(own_buf.at[0], ccw_buf, 1, ccw_send, ccw_recv, left))

        out_ref[pl.ds(me * m_per, half), :] = gemm(own_buf[0])
        out_ref[pl.ds(me * m_per + half, half), :] = gemm(own_buf[1])

        def wait_recv(buf, send_sems, recv_sems, slot, peer):
            rdma = pltpu.make_async_remote_copy(
                src_ref=buf.at[slot], dst_ref=buf.at[slot],
                send_sem=send_sems.at[slot], recv_sem=recv_sems.at[slot],
                device_id=(peer,), device_id_type=pl.DeviceIdType.MESH,
            )
            rdma.wait_recv()

        for s in range(N_PIECES):
            wait_recv(cw_buf, cw_send, cw_recv, s, right)
            if s + 2 < N_PIECES:
                started.append(
                    send(cw_buf.at[s], cw_buf, s + 2, cw_send, cw_recv, right))

            wait_recv(ccw_buf, ccw_send, ccw_recv, s, left)
            if s + 2 < N_PIECES:
                started.append(
                    send(ccw_buf.at[s], ccw_buf, s + 2, ccw_send, ccw_recv,
                         left))

            dist = s // 2 + 1
            origin_cw = pos_of_ring(my_r - dist)
            origin_ccw = pos_of_ring(my_r + dist)
            cw_off = 0 if s % 2 == 0 else half
            ccw_off = half if s % 2 == 0 else 0
            out_ref[pl.ds(origin_cw * m_per + cw_off, half), :] = gemm(
                cw_buf[s])
            out_ref[pl.ds(origin_ccw * m_per + ccw_off, half), :] = gemm(
                ccw_buf[s])

        for r in started:
            r.wait_send()

    out_shape = jax.ShapeDtypeStruct((N_DEV * m_per, n_per), jnp.float32)
    return pl.pallas_call(
        body,
        out_shape=out_shape,
        in_specs=[
            pl.BlockSpec(memory_space=pltpu.VMEM),
            pl.BlockSpec(memory_space=pltpu.VMEM),
        ],
        out_specs=pl.BlockSpec(memory_space=pltpu.VMEM),
        scratch_shapes=[
            pltpu.VMEM((2, half, k), jnp.bfloat16),
            pltpu.VMEM((k, n_per), jnp.bfloat16),
            pltpu.VMEM((N_PIECES, half, k), jnp.bfloat16),
            pltpu.VMEM((N_PIECES, half, k), jnp.bfloat16),
            pltpu.SemaphoreType.DMA((N_PIECES,)),
            pltpu.SemaphoreType.DMA((N_PIECES,)),
            pltpu.SemaphoreType.DMA((N_PIECES,)),
            pltpu.SemaphoreType.DMA((N_PIECES,)),
        ],
        compiler_params=pltpu.CompilerParams(
            collective_id=0,
            vmem_limit_bytes=100 * 1024 * 1024,
        ),
    )(x, w_mat)


# device time: 152742 ns/iter; 1.3836x vs baseline; 1.1462x over previous
import functools

import jax
import jax.numpy as jnp
from jax import lax
from jax.experimental import pallas as pl
from jax.experimental.pallas import tpu as pltpu

N_DEV = 8


def kernel(x, w_mat):
    m_per, k = x.shape
    _, n_per = w_mat.shape
    half = m_per // 2
    x = x.reshape(2, half, k)

    def body(x_ref, w_ref, out_ref, own_buf, w_bf, cw_buf, ccw_buf, par_buf,
             cw_send, cw_recv, ccw_send, ccw_recv, par_send, par_recv):
        me = lax.axis_index("i")

        def pos_of_ring(r):
            r = lax.rem(r + 2 * N_DEV, N_DEV)
            return jnp.where(r < 4, r, 11 - r)

        my_r = jnp.where(me < 4, me, 11 - me)
        right = pos_of_ring(my_r + 1)
        left = pos_of_ring(my_r - 1)
        is_even = lax.rem(my_r, 2) == 0
        partner = pos_of_ring(my_r + jnp.where(is_even, 3, -3))

        barrier_sem = pltpu.get_barrier_semaphore()
        for nbr in (left, right, partner):
            pl.semaphore_signal(
                barrier_sem, inc=1,
                device_id=(nbr,), device_id_type=pl.DeviceIdType.MESH,
            )
        own_buf[0] = x_ref[0].astype(jnp.bfloat16)
        own_buf[1] = x_ref[1].astype(jnp.bfloat16)
        w_bf[...] = w_ref[...].astype(jnp.bfloat16)
        pl.semaphore_wait(barrier_sem, 3)

        def gemm(piece):
            return lax.dot_general(
                piece, w_bf[...],
                (((1,), (0,)), ((), ())),
                preferred_element_type=jnp.float32,
            )

        def send(src, dst_buf, slot, send_sems, recv_sems, target):
            rdma = pltpu.make_async_remote_copy(
                src_ref=src, dst_ref=dst_buf.at[slot],
                send_sem=send_sems.at[slot], recv_sem=recv_sems.at[slot],
                device_id=(target,), device_id_type=pl.DeviceIdType.MESH,
            )
            rdma.start()
            return rdma

        def wait_recv(buf, send_sems, recv_sems, slot, peer):
            rdma = pltpu.make_async_remote_copy(
                src_ref=buf.at[slot], dst_ref=buf.at[slot],
                send_sem=send_sems.at[slot], recv_sem=recv_sems.at[slot],
                device_id=(peer,), device_id_type=pl.DeviceIdType.MESH,
            )
            rdma.wait_recv()

        send_cw = functools.partial(
            send, dst_buf=cw_buf, send_sems=cw_send, recv_sems=cw_recv,
            target=right)
        send_ccw = functools.partial(
            send, dst_buf=ccw_buf, send_sems=ccw_send, recv_sems=ccw_recv,
            target=left)
        send_par = functools.partial(
            send, dst_buf=par_buf, send_sems=par_send, recv_sems=par_recv,
            target=partner)

        def store(origin, off, piece_val):
            out_ref[pl.ds(origin * m_per + off, half), :] = gemm(piece_val)

        def branch(deep_cw):
            started = [
                send_cw(own_buf.at[0], slot=0),
                send_cw(own_buf.at[1], slot=1),
                send_ccw(own_buf.at[1], slot=0),
                send_ccw(own_buf.at[0], slot=1),
                send_par(own_buf.at[0], slot=0),
                send_par(own_buf.at[1], slot=1),
            ]
            store(me, 0, own_buf[0])
            store(me, half, own_buf[1])

            n_cw = 6 if deep_cw else 4
            n_ccw = 4 if deep_cw else 6
            fwd_cw_n = 4 if deep_cw else 6
            fwd_ccw_n = 6 if deep_cw else 4
            for t in range(6):
                if t < n_cw:
                    wait_recv(cw_buf, cw_send, cw_recv, t, right)
                    if t + 2 < fwd_cw_n:
                        started.append(send_cw(cw_buf.at[t], slot=t + 2))
                    if deep_cw and t < 2:
                        started.append(send_par(cw_buf.at[t], slot=t + 2))
                    dist = t // 2 + 1
                    store(pos_of_ring(my_r - dist),
                          0 if t % 2 == 0 else half, cw_buf[t])
                if t < n_ccw:
                    wait_recv(ccw_buf, ccw_send, ccw_recv, t, left)
                    if t + 2 < fwd_ccw_n:
                        started.append(send_ccw(ccw_buf.at[t], slot=t + 2))
                    if (not deep_cw) and t < 2:
                        started.append(send_par(ccw_buf.at[t], slot=3 - t))
                    dist = t // 2 + 1
                    store(pos_of_ring(my_r + dist),
                          half if t % 2 == 0 else 0, ccw_buf[t])
                if t < 4:
                    wait_recv(par_buf, par_send, par_recv, t, partner)
                    if t < 2:
                        o3 = pos_of_ring(my_r + jnp.where(is_even, 3, -3))
                        store(o3, 0 if t == 0 else half, par_buf[t])
                    else:
                        o4 = pos_of_ring(my_r + 4)
                        store(o4, 0 if t == 2 else half, par_buf[t])

            for r in started:
                r.wait_send()

        @pl.when(is_even)
        def _():
            branch(True)

        @pl.when(jnp.logical_not(is_even))
        def _():
            branch(False)

    out_shape = jax.ShapeDtypeStruct((N_DEV * m_per, n_per), jnp.float32)
    return pl.pallas_call(
        body,
        out_shape=out_shape,
        in_specs=[
            pl.BlockSpec(memory_space=pltpu.VMEM),
            pl.BlockSpec(memory_space=pltpu.VMEM),
        ],
        out_specs=pl.BlockSpec(memory_space=pltpu.VMEM),
        scratch_shapes=[
            pltpu.VMEM((2, half, k), jnp.bfloat16),
            pltpu.VMEM((k, n_per), jnp.bfloat16),
            pltpu.VMEM((6, half, k), jnp.bfloat16),
            pltpu.VMEM((6, half, k), jnp.bfloat16),
            pltpu.VMEM((4, half, k), jnp.bfloat16),
            pltpu.SemaphoreType.DMA((6,)),
            pltpu.SemaphoreType.DMA((6,)),
            pltpu.SemaphoreType.DMA((6,)),
            pltpu.SemaphoreType.DMA((6,)),
            pltpu.SemaphoreType.DMA((4,)),
            pltpu.SemaphoreType.DMA((4,)),
        ],
        compiler_params=pltpu.CompilerParams(
            collective_id=0,
            vmem_limit_bytes=100 * 1024 * 1024,
        ),
    )(x, w_mat)


# device time: 152740 ns/iter; 1.3836x vs baseline; 1.0000x over previous
import functools

import jax
import jax.numpy as jnp
from jax import lax
from jax.experimental import pallas as pl
from jax.experimental.pallas import tpu as pltpu

N_DEV = 8


def kernel(x, w_mat):
    m_per, k = x.shape
    _, n_per = w_mat.shape
    half = m_per // 2
    x = x.reshape(2, half, k)

    def body(x_ref, w_ref, out_ref, own_buf, w_bf, cw_buf, ccw_buf, par_buf,
             cw_send, cw_recv, ccw_send, ccw_recv, par_send, par_recv):
        me = lax.axis_index("i")

        def pos_of_ring(r):
            r = lax.rem(r + 2 * N_DEV, N_DEV)
            return jnp.where(r < 4, r, 11 - r)

        my_r = jnp.where(me < 4, me, 11 - me)
        right = pos_of_ring(my_r + 1)
        left = pos_of_ring(my_r - 1)
        is_even = lax.rem(my_r, 2) == 0
        partner = pos_of_ring(my_r + jnp.where(is_even, 3, -3))

        barrier_sem = pltpu.get_barrier_semaphore()
        for nbr in (left, right, partner):
            pl.semaphore_signal(
                barrier_sem, inc=1,
                device_id=(nbr,), device_id_type=pl.DeviceIdType.MESH,
            )
        own_buf[0] = x_ref[0].astype(jnp.bfloat16)
        pl.semaphore_wait(barrier_sem, 3)

        def gemm(piece):
            return lax.dot_general(
                piece, w_bf[...],
                (((1,), (0,)), ((), ())),
                preferred_element_type=jnp.float32,
            )

        def send(src, dst_buf, slot, send_sems, recv_sems, target):
            rdma = pltpu.make_async_remote_copy(
                src_ref=src, dst_ref=dst_buf.at[slot],
                send_sem=send_sems.at[slot], recv_sem=recv_sems.at[slot],
                device_id=(target,), device_id_type=pl.DeviceIdType.MESH,
            )
            rdma.start()
            return rdma

        def wait_recv(buf, send_sems, recv_sems, slot, peer):
            rdma = pltpu.make_async_remote_copy(
                src_ref=buf.at[slot], dst_ref=buf.at[slot],
                send_sem=send_sems.at[slot], recv_sem=recv_sems.at[slot],
                device_id=(peer,), device_id_type=pl.DeviceIdType.MESH,
            )
            rdma.wait_recv()

        send_cw = functools.partial(
            send, dst_buf=cw_buf, send_sems=cw_send, recv_sems=cw_recv,
            target=right)
        send_ccw = functools.partial(
            send, dst_buf=ccw_buf, send_sems=ccw_send, recv_sems=ccw_recv,
            target=left)
        send_par = functools.partial(
            send, dst_buf=par_buf, send_sems=par_send, recv_sems=par_recv,
            target=partner)

        def store(origin, off, piece_val):
            out_ref[pl.ds(origin * m_per + off, half), :] = gemm(piece_val)

        def branch(deep_cw):
            started = [
                send_cw(own_buf.at[0], slot=0),
                send_ccw(own_buf.at[0], slot=1),
                send_par(own_buf.at[0], slot=0),
            ]
            own_buf[1] = x_ref[1].astype(jnp.bfloat16)
            started += [
                send_cw(own_buf.at[1], slot=1),
                send_ccw(own_buf.at[1], slot=0),
                send_par(own_buf.at[1], slot=1),
            ]
            w_bf[...] = w_ref[...].astype(jnp.bfloat16)
            store(me, 0, own_buf[0])
            store(me, half, own_buf[1])

            n_cw = 6 if deep_cw else 4
            n_ccw = 4 if deep_cw else 6
            fwd_cw_n = 4 if deep_cw else 6
            fwd_ccw_n = 6 if deep_cw else 4
            for t in range(6):
                if t < n_cw:
                    wait_recv(cw_buf, cw_send, cw_recv, t, right)
                    if t + 2 < fwd_cw_n:
                        started.append(send_cw(cw_buf.at[t], slot=t + 2))
                    if deep_cw and t < 2:
                        started.append(send_par(cw_buf.at[t], slot=t + 2))
                    dist = t // 2 + 1
                    store(pos_of_ring(my_r - dist),
                          0 if t % 2 == 0 else half, cw_buf[t])
                if t < n_ccw:
                    wait_recv(ccw_buf, ccw_send, ccw_recv, t, left)
                    if t + 2 < fwd_ccw_n:
                        started.append(send_ccw(ccw_buf.at[t], slot=t + 2))
                    if (not deep_cw) and t < 2:
                        started.append(send_par(ccw_buf.at[t], slot=3 - t))
                    dist = t // 2 + 1
                    store(pos_of_ring(my_r + dist),
                          half if t % 2 == 0 else 0, ccw_buf[t])
                if t < 4:
                    wait_recv(par_buf, par_send, par_recv, t, partner)
                    if t < 2:
                        o3 = pos_of_ring(my_r + jnp.where(is_even, 3, -3))
                        store(o3, 0 if t == 0 else half, par_buf[t])
                    else:
                        o4 = pos_of_ring(my_r + 4)
                        store(o4, 0 if t == 2 else half, par_buf[t])

            for r in started:
                r.wait_send()

        @pl.when(is_even)
        def _():
            branch(True)

        @pl.when(jnp.logical_not(is_even))
        def _():
            branch(False)

    out_shape = jax.ShapeDtypeStruct((N_DEV * m_per, n_per), jnp.float32)
    return pl.pallas_call(
        body,
        out_shape=out_shape,
        in_specs=[
            pl.BlockSpec(memory_space=pltpu.VMEM),
            pl.BlockSpec(memory_space=pltpu.VMEM),
        ],
        out_specs=pl.BlockSpec(memory_space=pltpu.VMEM),
        scratch_shapes=[
            pltpu.VMEM((2, half, k), jnp.bfloat16),
            pltpu.VMEM((k, n_per), jnp.bfloat16),
            pltpu.VMEM((6, half, k), jnp.bfloat16),
            pltpu.VMEM((6, half, k), jnp.bfloat16),
            pltpu.VMEM((4, half, k), jnp.bfloat16),
            pltpu.SemaphoreType.DMA((6,)),
            pltpu.SemaphoreType.DMA((6,)),
            pltpu.SemaphoreType.DMA((6,)),
            pltpu.SemaphoreType.DMA((6,)),
            pltpu.SemaphoreType.DMA((4,)),
            pltpu.SemaphoreType.DMA((4,)),
        ],
        compiler_params=pltpu.CompilerParams(
            collective_id=0,
            vmem_limit_bytes=100 * 1024 * 1024,
        ),
    )(x, w_mat)
